# Initial kernel scaffold; baseline (speedup 1.0000x reference)
import jax
import jax.numpy as jnp
from jax.experimental import pallas as pl


def kernel(x, edge_index, edge_type, W1, root1, b1, W2, root2, b2):
    raise NotImplementedError("write your pallas kernel here")


# speedup vs baseline: 15.1731x; 15.1731x over previous
import functools

import jax
import jax.numpy as jnp
from jax import lax
from jax.experimental import pallas as pl
from jax.experimental.pallas import tpu as pltpu
from jax.experimental.pallas import tpu_sc as plsc

N = 10000
E = 160000
R = 8
DIN = 384
H = 64
DOUT = 3
W2P = 16
NR = N * R
NRP = NR + 16
NC = 2
NS = 16
NW = NC * NS
CHUNK = 128
NCH = 40
EPW = CHUNK * NCH
EPAD = EPW * NW
NP = 10240
RPS = NP // NS

_mesh = plsc.VectorSubcoreMesh(core_axis_name="c", subcore_axis_name="s")


@functools.partial(
    pl.kernel,
    out_type=jax.ShapeDtypeStruct((NW * NR,), jnp.float32),
    mesh=_mesh,
    scratch_types=[
        pltpu.VMEM((EPW,), jnp.int32),
        pltpu.VMEM((NRP,), jnp.float32),
    ],
    compiler_params=pltpu.CompilerParams(needs_layout_passes=False),
)
def _sc_counts(cw_hbm, out_hbm, cw_v, c_v):
    cid = lax.axis_index("c")
    sid = lax.axis_index("s")
    wid = cid * NS + sid

    zero16 = jnp.zeros((16,), jnp.float32)

    def zbody(i, carry):
        c_v[pl.ds(i * 16, 16)] = zero16
        return carry

    lax.fori_loop(0, NRP // 16, zbody, 0)

    pltpu.sync_copy(cw_hbm.at[wid], cw_v)
    ones16 = jnp.ones((16,), jnp.float32)

    def cbody(g, carry):
        idx = cw_v[pl.ds(g * 16, 16)]
        plsc.addupdate_scatter(c_v, [idx], ones16)
        return carry

    lax.fori_loop(0, EPW // 16, cbody, 0)

    pltpu.sync_copy(c_v.at[pl.ds(0, NR)], out_hbm.at[pl.ds(wid * NR, NR)])


def _make_sc_agg(width):
    nq = width // 16

    @functools.partial(
        pl.kernel,
        out_type=jax.ShapeDtypeStruct((NC, NP, width), jnp.float32),
        mesh=_mesh,
        scratch_types=[
            pltpu.VMEM((NCH, CHUNK), jnp.int32),
            pltpu.VMEM((NCH, CHUNK), jnp.int32),
            pltpu.VMEM((NCH, CHUNK), jnp.int32),
            pltpu.VMEM((CHUNK, width), jnp.float32),
            pltpu.VMEM((CHUNK, 16), jnp.float32),
            pltpu.VMEM_SHARED((NP, width), jnp.float32),
            pltpu.SemaphoreType.DMA,
            pltpu.SemaphoreType.DMA,
        ],
        compiler_params=pltpu.CompilerParams(needs_layout_passes=False,
                                             use_tc_tiling_on_sc=False),
    )
    def body(z_hbm, wtab_hbm, gidx_hbm, cwidx_hbm, didx_hbm, out_hbm,
             gidx_v, cwidx_v, didx_v, rows_v, wrows_v, acc_sh, sem, sem2):
        cid = lax.axis_index("c")
        sid = lax.axis_index("s")
        wid = cid * NS + sid

        zero16 = jnp.zeros((16,), jnp.float32)

        def zrow(i, carry):
            for q in range(nq):
                rows_v[i, pl.ds(q * 16, 16)] = zero16
            return carry

        lax.fori_loop(0, CHUNK, zrow, 0)
        base = sid * RPS
        for t in range(RPS // CHUNK):
            pltpu.sync_copy(rows_v,
                            acc_sh.at[pl.ds(base + t * CHUNK, CHUNK)])

        pltpu.sync_copy(gidx_hbm.at[wid], gidx_v)
        pltpu.sync_copy(cwidx_hbm.at[wid], cwidx_v)
        pltpu.sync_copy(didx_hbm.at[wid], didx_v)
        plsc.subcore_barrier()

        def chunk_body(ch, carry):
            cp1 = pltpu.async_copy(z_hbm.at[gidx_v.at[ch]], rows_v, sem)
            cp2 = pltpu.async_copy(wtab_hbm.at[cwidx_v.at[ch]], wrows_v, sem2)
            cp1.wait()
            cp2.wait()

            def scale(e, icarry):
                w = wrows_v[e, pl.ds(0, 16)][0]
                for q in range(nq):
                    rows_v[e, pl.ds(q * 16, 16)] = (
                        rows_v[e, pl.ds(q * 16, 16)] * w
                    )
                return icarry

            lax.fori_loop(0, CHUNK, scale, 0)
            pltpu.sync_copy(rows_v, acc_sh.at[didx_v.at[ch]], add=True)
            return carry

        lax.fori_loop(0, NCH, chunk_body, 0)
        plsc.subcore_barrier()

        for t in range(RPS // CHUNK):
            lo = base + t * CHUNK
            pltpu.sync_copy(acc_sh.at[pl.ds(lo, CHUNK)], rows_v)
            pltpu.sync_copy(rows_v, out_hbm.at[cid, pl.ds(lo, CHUNK)])

    return body


_sc_agg64 = _make_sc_agg(H)
_sc_agg16 = _make_sc_agg(W2P)


BN = 400
GRID = N // BN
CC = NR // GRID


def _tc1_body(x_ref, w1_ref, r1_ref, b1_ref, cnt_ref, z1_ref, base1_ref, wtab_ref):
    xb = x_ref[...]
    z1_ref[...] = jnp.dot(xb, w1_ref[...], preferred_element_type=jnp.float32)
    base1_ref[...] = (
        jnp.dot(xb, r1_ref[...], preferred_element_type=jnp.float32) + b1_ref[...]
    )
    csum = jnp.sum(cnt_ref[...], axis=0)
    cinv = 1.0 / jnp.maximum(csum, 1.0)
    wtab_ref[...] = jnp.broadcast_to(cinv[:, None], (CC, 16))


_tc1 = pl.pallas_call(
    _tc1_body,
    grid=(GRID,),
    in_specs=[
        pl.BlockSpec((BN, DIN), lambda i: (i, 0)),
        pl.BlockSpec((DIN, R * H), lambda i: (0, 0)),
        pl.BlockSpec((DIN, H), lambda i: (0, 0)),
        pl.BlockSpec((1, H), lambda i: (0, 0)),
        pl.BlockSpec((NW, CC), lambda i: (0, i)),
    ],
    out_specs=(
        pl.BlockSpec((BN, R * H), lambda i: (i, 0)),
        pl.BlockSpec((BN, H), lambda i: (i, 0)),
        pl.BlockSpec((CC, 16), lambda i: (i, 0)),
    ),
    out_shape=(
        jax.ShapeDtypeStruct((N, R * H), jnp.float32),
        jax.ShapeDtypeStruct((N, H), jnp.float32),
        jax.ShapeDtypeStruct((NR, 16), jnp.float32),
    ),
)


def _tc2_body(b1_ref, agg_ref, w2_ref, r2_ref, b2_ref, z2_ref, base2_ref):
    h = jnp.maximum(b1_ref[...] + agg_ref[0] + agg_ref[1], 0.0)
    z2_ref[...] = jnp.dot(h, w2_ref[...], preferred_element_type=jnp.float32)
    base2_ref[...] = (
        jnp.dot(h, r2_ref[...], preferred_element_type=jnp.float32) + b2_ref[...]
    )


_tc2 = pl.pallas_call(
    _tc2_body,
    grid=(GRID,),
    in_specs=[
        pl.BlockSpec((BN, H), lambda i: (i, 0)),
        pl.BlockSpec((NC, BN, H), lambda i: (0, i, 0)),
        pl.BlockSpec((H, R * W2P), lambda i: (0, 0)),
        pl.BlockSpec((H, W2P), lambda i: (0, 0)),
        pl.BlockSpec((1, W2P), lambda i: (0, 0)),
    ],
    out_specs=(
        pl.BlockSpec((BN, R * W2P), lambda i: (i, 0)),
        pl.BlockSpec((BN, W2P), lambda i: (i, 0)),
    ),
    out_shape=(
        jax.ShapeDtypeStruct((N, R * W2P), jnp.float32),
        jax.ShapeDtypeStruct((N, W2P), jnp.float32),
    ),
)


def _tc3_body(b2_ref, agg_ref, out_ref):
    out_ref[...] = b2_ref[...] + agg_ref[0] + agg_ref[1]


_tc3 = pl.pallas_call(
    _tc3_body,
    grid=(GRID,),
    in_specs=[
        pl.BlockSpec((BN, W2P), lambda i: (i, 0)),
        pl.BlockSpec((NC, BN, W2P), lambda i: (0, i, 0)),
    ],
    out_specs=pl.BlockSpec((BN, W2P), lambda i: (i, 0)),
    out_shape=jax.ShapeDtypeStruct((N, W2P), jnp.float32),
)


def _impl(x, edge_index, edge_type, W1, root1, b1, W2, root2, b2):
    src = edge_index[0]
    dst = edge_index[1]
    et = edge_type

    pad = EPAD - E
    gidx = jnp.concatenate([src * R + et, jnp.zeros((pad,), jnp.int32)])
    cwidx = jnp.concatenate([dst * R + et, jnp.full((pad,), NR, jnp.int32)])
    didx = jnp.concatenate([dst, jnp.zeros((pad,), jnp.int32)])
    gidx3 = gidx.reshape(NW, NCH, CHUNK)
    cwidx3 = cwidx.reshape(NW, NCH, CHUNK)
    didx3 = didx.reshape(NW, NCH, CHUNK)

    counts = _sc_counts(cwidx.reshape(NW, EPW)).reshape(NW, NR)

    W1cat = jnp.transpose(W1, (1, 0, 2)).reshape(DIN, R * H)
    z1, base1, wtab_g = _tc1(x, W1cat, root1, b1.reshape(1, H), counts)
    wtab = jnp.concatenate([wtab_g, jnp.zeros((16, 16), jnp.float32)])

    aggs1 = _sc_agg64(z1.reshape(NR, H), wtab, gidx3, cwidx3, didx3)

    W2p = jnp.pad(jnp.transpose(W2, (1, 0, 2)),
                  ((0, 0), (0, 0), (0, W2P - DOUT))).reshape(H, R * W2P)
    root2p = jnp.pad(root2, ((0, 0), (0, W2P - DOUT)))
    b2p = jnp.pad(b2, (0, W2P - DOUT)).reshape(1, W2P)
    z2, base2 = _tc2(base1, aggs1, W2p, root2p, b2p)

    aggs2 = _sc_agg16(z2.reshape(NR, W2P), wtab, gidx3, cwidx3, didx3)

    out16 = _tc3(base2, aggs2)
    return out16[:, :DOUT]

    W1cat = jnp.transpose(W1, (1, 0, 2)).reshape(DIN, R * H)
    z1, base1, wtab_g = _tc1(x, W1cat, root1, b1.reshape(1, H), counts)
    wtab = jnp.concatenate([wtab_g, jnp.zeros((16, 16), jnp.float32)])

    aggs1 = _sc_agg64(z1.reshape(NR, H), wtab, gidx3, cwidx3, didx3)

    W2p = jnp.pad(jnp.transpose(W2, (1, 0, 2)),
                  ((0, 0), (0, 0), (0, W2P - DOUT))).reshape(H, R * W2P)
    root2p = jnp.pad(root2, ((0, 0), (0, W2P - DOUT)))
    b2p = jnp.pad(b2, (0, W2P - DOUT)).reshape(1, W2P)
    z2, base2 = _tc2(base1, aggs1, W2p, root2p, b2p)

    aggs2 = _sc_agg16(z2.reshape(NR, W2P), wtab, gidx3, cwidx3, didx3)

    out16 = _tc3(base2, aggs2)
    return out16[:, :DOUT]


kernel = jax.jit(_impl)


# speedup vs baseline: 25.0998x; 1.6542x over previous
import functools

import jax
import jax.numpy as jnp
from jax import lax
from jax.experimental import pallas as pl
from jax.experimental.pallas import tpu as pltpu
from jax.experimental.pallas import tpu_sc as plsc

N = 10000
E = 160000
R = 8
DIN = 384
H = 64
DOUT = 3
W2P = 16
NR = N * R
NC = 2
NS = 16
NW = NC * NS
NCH = 40
NPAIR = NCH // 2
CHUNK = 125
EPW = CHUNK * NCH
ZCH = 128
NP = 10240
RPS = NP // NS

_mesh = plsc.VectorSubcoreMesh(core_axis_name="c", subcore_axis_name="s")


@functools.partial(
    pl.kernel,
    out_type=jax.ShapeDtypeStruct((NW * NR,), jnp.float32),
    mesh=_mesh,
    scratch_types=[
        pltpu.VMEM((EPW,), jnp.int32),
        pltpu.VMEM((NR,), jnp.float32),
    ],
    compiler_params=pltpu.CompilerParams(needs_layout_passes=False),
)
def _sc_counts(cw_hbm, out_hbm, cw_v, c_v):
    cid = lax.axis_index("c")
    sid = lax.axis_index("s")
    wid = cid * NS + sid

    zero16 = jnp.zeros((16,), jnp.float32)

    def zbody(i, carry):
        c_v[pl.ds(i * 16, 16)] = zero16
        return carry

    lax.fori_loop(0, NR // 16, zbody, 0)

    pltpu.sync_copy(cw_hbm.at[wid], cw_v)
    ones16 = jnp.ones((16,), jnp.float32)

    def cbody(g, carry):
        idx = cw_v[pl.ds(g * 16, 16)]
        plsc.addupdate_scatter(c_v, [idx], ones16)
        return carry

    lax.fori_loop(0, EPW // 16, cbody, 0)
    rem = EPW - (EPW // 16) * 16
    if rem:
        idx = cw_v[pl.ds(EPW - 16, 16)]
        mask = lax.iota(jnp.int32, 16) >= (16 - rem)
        plsc.addupdate_scatter(c_v, [idx], ones16, mask=mask)

    pltpu.sync_copy(c_v, out_hbm.at[pl.ds(wid * NR, NR)])


def _make_sc_agg(width):
    nq = width // 16

    @functools.partial(
        pl.kernel,
        out_type=jax.ShapeDtypeStruct((NC, NP, width), jnp.float32),
        mesh=_mesh,
        scratch_types=[
            pltpu.VMEM((NCH, CHUNK), jnp.int32),
            pltpu.VMEM((NCH, CHUNK), jnp.int32),
            pltpu.VMEM((NCH, CHUNK), jnp.int32),
            pltpu.VMEM((ZCH, width), jnp.float32),
            pltpu.VMEM((ZCH, width), jnp.float32),
            pltpu.VMEM((CHUNK, 16), jnp.float32),
            pltpu.VMEM((CHUNK, 16), jnp.float32),
            pltpu.VMEM_SHARED((NP, width), jnp.float32),
            pltpu.SemaphoreType.DMA,
            pltpu.SemaphoreType.DMA,
            pltpu.SemaphoreType.DMA,
            pltpu.SemaphoreType.DMA,
        ],
        compiler_params=pltpu.CompilerParams(needs_layout_passes=False,
                                             use_tc_tiling_on_sc=False),
    )
    def body(z_hbm, wtab_hbm, gidx_hbm, cwidx_hbm, didx_hbm, out_hbm,
             gidx_v, cwidx_v, didx_v, rows_a, rows_b, wrows_a, wrows_b,
             acc_sh, sza, szb, swa, swb):
        cid = lax.axis_index("c")
        sid = lax.axis_index("s")
        wid = cid * NS + sid

        zero16 = jnp.zeros((16,), jnp.float32)

        def zrow(i, carry):
            for q in range(nq):
                rows_a[i, pl.ds(q * 16, 16)] = zero16
            return carry

        lax.fori_loop(0, ZCH, zrow, 0)
        base = sid * RPS
        for t in range(RPS // ZCH):
            pltpu.sync_copy(rows_a,
                            acc_sh.at[pl.ds(base + t * ZCH, ZCH)])

        pltpu.sync_copy(gidx_hbm.at[wid], gidx_v)
        pltpu.sync_copy(cwidx_hbm.at[wid], cwidx_v)
        pltpu.sync_copy(didx_hbm.at[wid], didx_v)
        plsc.subcore_barrier()

        def issue(ch, rv, wv, s1, s2):
            pltpu.async_copy(z_hbm.at[gidx_v.at[ch]],
                             rv.at[pl.ds(0, CHUNK)], s1)
            pltpu.async_copy(wtab_hbm.at[cwidx_v.at[ch]], wv, s2)

        def wait(rv, wv, s1, s2):
            pltpu.make_async_copy(z_hbm.at[gidx_v.at[0]],
                                  rv.at[pl.ds(0, CHUNK)], s1).wait()
            pltpu.make_async_copy(wtab_hbm.at[cwidx_v.at[0]], wv, s2).wait()

        def scale_scatter(ch, rv, wv):
            def scale(e, icarry):
                w = wv[e, pl.ds(0, 16)][0]
                for q in range(nq):
                    rv[e, pl.ds(q * 16, 16)] = rv[e, pl.ds(q * 16, 16)] * w
                return icarry

            lax.fori_loop(0, CHUNK, scale, 0)
            pltpu.sync_copy(rv.at[pl.ds(0, CHUNK)],
                            acc_sh.at[didx_v.at[ch]], add=True)

        issue(0, rows_a, wrows_a, sza, swa)

        def pair(p, carry):
            ch0 = 2 * p
            issue(ch0 + 1, rows_b, wrows_b, szb, swb)
            wait(rows_a, wrows_a, sza, swa)
            scale_scatter(ch0, rows_a, wrows_a)

            @pl.when(p < NPAIR - 1)
            def _():
                issue(ch0 + 2, rows_a, wrows_a, sza, swa)

            wait(rows_b, wrows_b, szb, swb)
            scale_scatter(ch0 + 1, rows_b, wrows_b)
            return carry

        lax.fori_loop(0, NPAIR, pair, 0)
        plsc.subcore_barrier()

        for t in range(RPS // ZCH):
            lo = base + t * ZCH
            pltpu.sync_copy(acc_sh.at[pl.ds(lo, ZCH)], rows_a)
            pltpu.sync_copy(rows_a, out_hbm.at[cid, pl.ds(lo, ZCH)])

    return body


_sc_agg64 = _make_sc_agg(H)
_sc_agg16 = _make_sc_agg(W2P)


BN = 400
GRID = N // BN
CC = NR // GRID


def _tc1a_body(x_ref, w1_ref, r1_ref, b1_ref, z1_ref, base1_ref):
    xb = x_ref[...]
    z1_ref[...] = jnp.dot(xb, w1_ref[...], preferred_element_type=jnp.float32)
    base1_ref[...] = (
        jnp.dot(xb, r1_ref[...], preferred_element_type=jnp.float32) + b1_ref[...]
    )


_tc1a = pl.pallas_call(
    _tc1a_body,
    grid=(GRID,),
    in_specs=[
        pl.BlockSpec((BN, DIN), lambda i: (i, 0)),
        pl.BlockSpec((DIN, R * H), lambda i: (0, 0)),
        pl.BlockSpec((DIN, H), lambda i: (0, 0)),
        pl.BlockSpec((1, H), lambda i: (0, 0)),
    ],
    out_specs=(
        pl.BlockSpec((BN, R * H), lambda i: (i, 0)),
        pl.BlockSpec((BN, H), lambda i: (i, 0)),
    ),
    out_shape=(
        jax.ShapeDtypeStruct((N, R * H), jnp.float32),
        jax.ShapeDtypeStruct((N, H), jnp.float32),
    ),
)


def _tc1b_body(cnt_ref, wtab_ref):
    csum = jnp.sum(cnt_ref[...], axis=0)
    cinv = 1.0 / jnp.maximum(csum, 1.0)
    wtab_ref[...] = jnp.broadcast_to(cinv[:, None], (CC, 16))


_tc1b = pl.pallas_call(
    _tc1b_body,
    grid=(GRID,),
    in_specs=[pl.BlockSpec((NW, CC), lambda i: (0, i))],
    out_specs=pl.BlockSpec((CC, 16), lambda i: (i, 0)),
    out_shape=jax.ShapeDtypeStruct((NR, 16), jnp.float32),
)


def _tc2_body(b1_ref, agg_ref, w2_ref, r2_ref, b2_ref, z2_ref, base2_ref):
    h = jnp.maximum(b1_ref[...] + agg_ref[0] + agg_ref[1], 0.0)
    z2_ref[...] = jnp.dot(h, w2_ref[...], preferred_element_type=jnp.float32)
    base2_ref[...] = (
        jnp.dot(h, r2_ref[...], preferred_element_type=jnp.float32) + b2_ref[...]
    )


_tc2 = pl.pallas_call(
    _tc2_body,
    grid=(GRID,),
    in_specs=[
        pl.BlockSpec((BN, H), lambda i: (i, 0)),
        pl.BlockSpec((NC, BN, H), lambda i: (0, i, 0)),
        pl.BlockSpec((H, R * W2P), lambda i: (0, 0)),
        pl.BlockSpec((H, W2P), lambda i: (0, 0)),
        pl.BlockSpec((1, W2P), lambda i: (0, 0)),
    ],
    out_specs=(
        pl.BlockSpec((BN, R * W2P), lambda i: (i, 0)),
        pl.BlockSpec((BN, W2P), lambda i: (i, 0)),
    ),
    out_shape=(
        jax.ShapeDtypeStruct((N, R * W2P), jnp.float32),
        jax.ShapeDtypeStruct((N, W2P), jnp.float32),
    ),
)


def _tc3_body(b2_ref, agg_ref, out_ref):
    out_ref[...] = b2_ref[...] + agg_ref[0] + agg_ref[1]


_tc3 = pl.pallas_call(
    _tc3_body,
    grid=(GRID,),
    in_specs=[
        pl.BlockSpec((BN, W2P), lambda i: (i, 0)),
        pl.BlockSpec((NC, BN, W2P), lambda i: (0, i, 0)),
    ],
    out_specs=pl.BlockSpec((BN, W2P), lambda i: (i, 0)),
    out_shape=jax.ShapeDtypeStruct((N, W2P), jnp.float32),
)


def _impl(x, edge_index, edge_type, W1, root1, b1, W2, root2, b2):
    src = edge_index[0]
    dst = edge_index[1]
    et = edge_type

    gidx3 = (src * R + et).reshape(NW, NCH, CHUNK)
    cwidx = dst * R + et
    cwidx3 = cwidx.reshape(NW, NCH, CHUNK)
    didx3 = dst.reshape(NW, NCH, CHUNK)

    counts = _sc_counts(cwidx.reshape(NW, EPW)).reshape(NW, NR)

    W1cat = jnp.transpose(W1, (1, 0, 2)).reshape(DIN, R * H)
    z1, base1 = _tc1a(x, W1cat, root1, b1.reshape(1, H))
    wtab = _tc1b(counts)

    aggs1 = _sc_agg64(z1.reshape(NR, H), wtab, gidx3, cwidx3, didx3)

    W2p = jnp.pad(jnp.transpose(W2, (1, 0, 2)),
                  ((0, 0), (0, 0), (0, W2P - DOUT))).reshape(H, R * W2P)
    root2p = jnp.pad(root2, ((0, 0), (0, W2P - DOUT)))
    b2p = jnp.pad(b2, (0, W2P - DOUT)).reshape(1, W2P)
    z2, base2 = _tc2(base1, aggs1, W2p, root2p, b2p)

    aggs2 = _sc_agg16(z2.reshape(NR, W2P), wtab, gidx3, cwidx3, didx3)

    out16 = _tc3(base2, aggs2)
    return out16[:, :DOUT]


kernel = jax.jit(_impl)
